# Initial kernel scaffold; baseline (speedup 1.0000x reference)
import jax
import jax.numpy as jnp
from jax.experimental import pallas as pl


def kernel(coords, features, t_embed, params1, params2):
    raise NotImplementedError("write your pallas kernel here")


# speedup vs baseline: 456.7993x; 456.7993x over previous
import functools

import jax
import jax.numpy as jnp
from jax import lax
from jax.experimental import pallas as pl
from jax.experimental.pallas import tpu as pltpu
from jax.experimental.pallas import tpu_sc as plsc

_B = 4
_N = 8192
_M = 1024
_RADII = (0.2, 0.4)
_NSAMPLES = (16, 32)

_NUM_CORES = 2
_NUM_SUBCORES = 16
_NW = _NUM_CORES * _NUM_SUBCORES
_QPW = (_B * _M) // _NW

_F32 = jnp.float32
_I32 = jnp.int32


def _sc_mesh():
    return plsc.VectorSubcoreMesh(
        core_axis_name="c", subcore_axis_name="s",
        num_cores=_NUM_CORES, num_subcores=_NUM_SUBCORES)


def _fps_body(cx_ref, cy_ref, cz_ref, sx_ref, sy_ref, sz_ref, dist_ref):
    lanes = lax.broadcasted_iota(_I32, (8, _N), 1)
    out_lanes = lax.broadcasted_iota(_I32, (8, _M), 1)
    cx = cx_ref[...]
    cy = cy_ref[...]
    cz = cz_ref[...]

    def extract(c, nxt):
        return jnp.sum(jnp.where(lanes == nxt, c, 0.0), axis=1, keepdims=True)

    def step(i, nxt):
        px = extract(cx, nxt)
        py = extract(cy, nxt)
        pz = extract(cz, nxt)
        m = out_lanes == i
        sx_ref[...] = jnp.where(m, px, sx_ref[...])
        sy_ref[...] = jnp.where(m, py, sy_ref[...])
        sz_ref[...] = jnp.where(m, pz, sz_ref[...])
        dx = cx - px
        dy = cy - py
        dz = cz - pz
        return dx * dx + dy * dy + dz * dz

    dist_ref[...] = step(0, jnp.zeros((8, 1), _I32))

    def body(i, carry):
        dist = dist_ref[...]
        mx = jnp.max(dist, axis=1, keepdims=True)
        nxt = jnp.min(jnp.where(dist == mx, lanes, _N), axis=1, keepdims=True)
        d2 = step(i, nxt)
        dist_ref[...] = jnp.minimum(dist, d2)
        return carry

    lax.fori_loop(1, _M, body, 0)


def _fps(cx, cy, cz):
    return pl.pallas_call(
        _fps_body,
        out_shape=[jax.ShapeDtypeStruct((8, _M), _F32)] * 3,
        scratch_shapes=[pltpu.VMEM((8, _N), _F32)],
    )(cx, cy, cz)


def _ytab_body(x_ref, w1_ref, w2_ref, y1_ref, y2_ref):
    x = x_ref[...]
    y1_ref[...] = jnp.dot(x, w1_ref[...], preferred_element_type=_F32,
                          precision=lax.Precision.HIGHEST)
    y2_ref[...] = jnp.dot(x, w2_ref[...], preferred_element_type=_F32,
                          precision=lax.Precision.HIGHEST)


def _ytab(featT, w1t, w2t):
    rows = featT.shape[0]
    rb = 1024
    c1 = w1t.shape[1]
    c2 = w2t.shape[1]
    return pl.pallas_call(
        _ytab_body,
        grid=(rows // rb,),
        in_specs=[
            pl.BlockSpec((rb, 64), lambda i: (i, 0)),
            pl.BlockSpec((64, c1), lambda i: (0, 0)),
            pl.BlockSpec((64, c2), lambda i: (0, 0)),
        ],
        out_specs=[
            pl.BlockSpec((rb, c1), lambda i: (i, 0)),
            pl.BlockSpec((rb, c2), lambda i: (i, 0)),
        ],
        out_shape=[jax.ShapeDtypeStruct((rows, c1), _F32),
                   jax.ShapeDtypeStruct((rows, c2), _F32)],
    )(featT, w1t, w2t)


def _ball_query(cxf, cyf, czf, qx, qy, qz):
    s1, s2 = _NSAMPLES
    r1 = _B * _M * s1
    r2 = _B * _M * s2
    wpb = _NW // _B

    @functools.partial(
        pl.kernel,
        mesh=_sc_mesh(),
        compiler_params=pltpu.CompilerParams(needs_layout_passes=False),
        out_type=[
            jax.ShapeDtypeStruct((r1,), _I32),
            jax.ShapeDtypeStruct((r2,), _I32),
            jax.ShapeDtypeStruct((r1,), _F32),
            jax.ShapeDtypeStruct((r1,), _F32),
            jax.ShapeDtypeStruct((r1,), _F32),
            jax.ShapeDtypeStruct((r2,), _F32),
            jax.ShapeDtypeStruct((r2,), _F32),
            jax.ShapeDtypeStruct((r2,), _F32),
        ],
        scratch_types=[
            pltpu.VMEM((_N,), _F32),
            pltpu.VMEM((_N,), _F32),
            pltpu.VMEM((_N,), _F32),
            pltpu.VMEM((_QPW,), _F32),
            pltpu.VMEM((_QPW,), _F32),
            pltpu.VMEM((_QPW,), _F32),
            pltpu.VMEM((_QPW * s1,), _I32),
            pltpu.VMEM((_QPW * s2,), _I32),
            pltpu.VMEM((_QPW * s1,), _F32),
            pltpu.VMEM((_QPW * s1,), _F32),
            pltpu.VMEM((_QPW * s1,), _F32),
            pltpu.VMEM((_QPW * s2,), _F32),
            pltpu.VMEM((_QPW * s2,), _F32),
            pltpu.VMEM((_QPW * s2,), _F32),
            pltpu.VMEM((s1 + 16,), _I32),
            pltpu.VMEM((s2 + 16,), _I32),
        ],
    )
    def ball(cx_h, cy_h, cz_h, qx_h, qy_h, qz_h,
             i1_h, i2_h, rx1_h, ry1_h, rz1_h, rx2_h, ry2_h, rz2_h,
             cx_v, cy_v, cz_v, qx_v, qy_v, qz_v,
             i1_v, i2_v, rx1_v, ry1_v, rz1_v, rx2_v, ry2_v, rz2_v,
             buf1_v, buf2_v):
        wid = lax.axis_index("s") * _NUM_CORES + lax.axis_index("c")
        b = wid // wpb
        qbase = b * _M + (wid % wpb) * _QPW
        pltpu.sync_copy(cx_h.at[pl.ds(b * _N, _N)], cx_v)
        pltpu.sync_copy(cy_h.at[pl.ds(b * _N, _N)], cy_v)
        pltpu.sync_copy(cz_h.at[pl.ds(b * _N, _N)], cz_v)
        pltpu.sync_copy(qx_h.at[pl.ds(qbase, _QPW)], qx_v)
        pltpu.sync_copy(qy_h.at[pl.ds(qbase, _QPW)], qy_v)
        pltpu.sync_copy(qz_h.at[pl.ds(qbase, _QPW)], qz_v)

        lane = lax.iota(_I32, 16)
        zero16 = jnp.zeros((16,), _I32)

        def one_query(i, pqx, pqy, pqz, S, rad2, buf_v, iv_v, rxv, ryv, rzv):
            def cond(st):
                c, cnt = st
                return jnp.logical_and(cnt < S, c < _N // 16)

            def bodyw(st):
                c, cnt = st
                px = cx_v[pl.ds(c * 16, 16)]
                py = cy_v[pl.ds(c * 16, 16)]
                pz = cz_v[pl.ds(c * 16, 16)]
                dx = px - pqx
                dy = py - pqy
                dz = pz - pqz
                d2 = dx * dx + dy * dy + dz * dz
                msk = d2 <= rad2
                plsc.store_compressed(buf_v.at[pl.ds(cnt, 16)],
                                      lane + c * 16, mask=msk)
                npos = jnp.max(plsc.all_reduce_population_count(msk))
                return c + 1, cnt + npos

            _, cnt = lax.while_loop(cond, bodyw,
                                    (jnp.int32(0), jnp.int32(0)))
            first = plsc.load_gather(buf_v, [zero16])
            for k in range(S // 16):
                v = buf_v[pl.ds(k * 16, 16)]
                slot = lane + k * 16
                padded = jnp.where(slot < cnt, v, first)
                iv_v[pl.ds(i * S + k * 16, 16)] = padded + b * _N
                rxv[pl.ds(i * S + k * 16, 16)] = (
                    plsc.load_gather(cx_v, [padded]) - pqx)
                ryv[pl.ds(i * S + k * 16, 16)] = (
                    plsc.load_gather(cy_v, [padded]) - pqy)
                rzv[pl.ds(i * S + k * 16, 16)] = (
                    plsc.load_gather(cz_v, [padded]) - pqz)

        def qbody(i, carry):
            iv = zero16 + i
            pqx = plsc.load_gather(qx_v, [iv])
            pqy = plsc.load_gather(qy_v, [iv])
            pqz = plsc.load_gather(qz_v, [iv])
            one_query(i, pqx, pqy, pqz, s1, _RADII[0] * _RADII[0],
                      buf1_v, i1_v, rx1_v, ry1_v, rz1_v)
            one_query(i, pqx, pqy, pqz, s2, _RADII[1] * _RADII[1],
                      buf2_v, i2_v, rx2_v, ry2_v, rz2_v)
            return carry

        lax.fori_loop(0, _QPW, qbody, 0)

        pltpu.sync_copy(i1_v, i1_h.at[pl.ds(qbase * s1, _QPW * s1)])
        pltpu.sync_copy(i2_v, i2_h.at[pl.ds(qbase * s2, _QPW * s2)])
        pltpu.sync_copy(rx1_v, rx1_h.at[pl.ds(qbase * s1, _QPW * s1)])
        pltpu.sync_copy(ry1_v, ry1_h.at[pl.ds(qbase * s1, _QPW * s1)])
        pltpu.sync_copy(rz1_v, rz1_h.at[pl.ds(qbase * s1, _QPW * s1)])
        pltpu.sync_copy(rx2_v, rx2_h.at[pl.ds(qbase * s2, _QPW * s2)])
        pltpu.sync_copy(ry2_v, ry2_h.at[pl.ds(qbase * s2, _QPW * s2)])
        pltpu.sync_copy(rz2_v, rz2_h.at[pl.ds(qbase * s2, _QPW * s2)])

    return ball(cxf, cyf, czf, qx, qy, qz)


def _gather(ytab, ttab, idx, cf):
    rows = idx.shape[0]
    rpw = rows // _NW
    ch = 128

    @functools.partial(
        pl.kernel,
        mesh=_sc_mesh(),
        compiler_params=pltpu.CompilerParams(needs_layout_passes=False,
                                             use_tc_tiling_on_sc=False),
        out_type=[jax.ShapeDtypeStruct((rows, cf), _F32),
                  jax.ShapeDtypeStruct((rows, 64), _F32)],
        scratch_types=[
            pltpu.VMEM((rpw,), _I32),
            pltpu.VMEM((ch, cf), _F32),
            pltpu.VMEM((ch, 64), _F32),
            pltpu.SemaphoreType.DMA,
            pltpu.SemaphoreType.DMA,
        ],
    )
    def gk(ytab_h, ttab_h, idx_h, g_h, t_h, idx_v, yv, tv, sem1, sem2):
        wid = lax.axis_index("s") * _NUM_CORES + lax.axis_index("c")
        base = wid * rpw
        pltpu.sync_copy(idx_h.at[pl.ds(base, rpw)], idx_v)

        def body(k, carry):
            ii = idx_v.at[pl.ds(k * ch, ch)]
            c1 = pltpu.async_copy(ytab_h.at[ii], yv, sem1)
            c2 = pltpu.async_copy(ttab_h.at[ii], tv, sem2)
            c1.wait()
            c2.wait()
            pltpu.sync_copy(yv, g_h.at[pl.ds(base + k * ch, ch)])
            pltpu.sync_copy(tv, t_h.at[pl.ds(base + k * ch, ch)])
            return carry

        lax.fori_loop(0, rpw // ch, body, 0)

    return gk(ytab, ttab, idx)


def _mlp_a(g, rx, ry, rz, wcr, pb, c1):
    rows = g.shape[0]
    rb = 1024

    def body(g_ref, rx_ref, ry_ref, rz_ref, wc_ref, pb_ref, z_ref, st_ref):
        i = pl.program_id(0)
        wc = wc_ref[...]
        z = (g_ref[...]
             + rx_ref[...] * wc[0:1, :]
             + ry_ref[...] * wc[1:2, :]
             + rz_ref[...] * wc[2:3, :]
             + pb_ref[0:1, :])
        z_ref[...] = z
        s = jnp.sum(z, axis=0, keepdims=True)
        ss = jnp.sum(z * z, axis=0, keepdims=True)
        acc = jnp.concatenate([s, ss, jnp.zeros((6, c1), _F32)], axis=0)

        @pl.when(i == 0)
        def _():
            st_ref[...] = acc

        @pl.when(i != 0)
        def _():
            st_ref[...] = st_ref[...] + acc

    return pl.pallas_call(
        body,
        grid=(rows // rb,),
        in_specs=[
            pl.BlockSpec((rb, c1), lambda i: (i, 0)),
            pl.BlockSpec((rb, 1), lambda i: (i, 0)),
            pl.BlockSpec((rb, 1), lambda i: (i, 0)),
            pl.BlockSpec((rb, 1), lambda i: (i, 0)),
            pl.BlockSpec((8, c1), lambda i: (0, 0)),
            pl.BlockSpec((8, c1), lambda i: (0, 0)),
        ],
        out_specs=[
            pl.BlockSpec((rb, c1), lambda i: (i, 0)),
            pl.BlockSpec((8, c1), lambda i: (0, 0)),
        ],
        out_shape=[jax.ShapeDtypeStruct((rows, c1), _F32),
                   jax.ShapeDtypeStruct((8, c1), _F32)],
    )(g, rx, ry, rz, wcr, pb)


def _bn_coeffs(st, pb, rows):
    rinv = 1.0 / rows
    mean = st[0:1, :] * rinv
    var = st[1:2, :] * rinv - mean * mean
    a = lax.rsqrt(var + 1e-5) * pb[1:2, :]
    c = pb[2:3, :] - mean * a
    return a, c


def _mlp_b(z1, st1, pb1, w2t, pb2, c1, c2):
    rows = z1.shape[0]
    rb = 1024

    def body(z_ref, st_ref, pb1_ref, w2_ref, pb2_ref, z2_ref, st2_ref):
        i = pl.program_id(0)
        a, c = _bn_coeffs(st_ref[...], pb1_ref[...], rows)
        x1 = jnp.maximum(z_ref[...] * a + c, 0.0)
        z2 = jnp.dot(x1, w2_ref[...], preferred_element_type=_F32,
                     precision=lax.Precision.HIGHEST) + pb2_ref[0:1, :]
        z2_ref[...] = z2
        s = jnp.sum(z2, axis=0, keepdims=True)
        ss = jnp.sum(z2 * z2, axis=0, keepdims=True)
        acc = jnp.concatenate([s, ss, jnp.zeros((6, c2), _F32)], axis=0)

        @pl.when(i == 0)
        def _():
            st2_ref[...] = acc

        @pl.when(i != 0)
        def _():
            st2_ref[...] = st2_ref[...] + acc

    return pl.pallas_call(
        body,
        grid=(rows // rb,),
        in_specs=[
            pl.BlockSpec((rb, c1), lambda i: (i, 0)),
            pl.BlockSpec((8, c1), lambda i: (0, 0)),
            pl.BlockSpec((8, c1), lambda i: (0, 0)),
            pl.BlockSpec((c1, c2), lambda i: (0, 0)),
            pl.BlockSpec((8, c2), lambda i: (0, 0)),
        ],
        out_specs=[
            pl.BlockSpec((rb, c2), lambda i: (i, 0)),
            pl.BlockSpec((8, c2), lambda i: (0, 0)),
        ],
        out_shape=[jax.ShapeDtypeStruct((rows, c2), _F32),
                   jax.ShapeDtypeStruct((8, c2), _F32)],
    )(z1, st1, pb1, w2t, pb2)


def _mlp_c(z2r, st2, pb2, tr, s, c2):
    bm = z2r.shape[0]
    rows = bm * s
    qb = 64

    def body(z_ref, st_ref, pb_ref, t_ref, f_ref, to_ref):
        a, c = _bn_coeffs(st_ref[...], pb_ref[...], rows)
        x2 = jnp.maximum(z_ref[...] * a[None] + c[None], 0.0)
        f_ref[...] = jnp.max(x2, axis=1)
        to_ref[...] = jnp.max(t_ref[...], axis=1)

    return pl.pallas_call(
        body,
        grid=(bm // qb,),
        in_specs=[
            pl.BlockSpec((qb, s, c2), lambda q: (q, 0, 0)),
            pl.BlockSpec((8, c2), lambda q: (0, 0)),
            pl.BlockSpec((8, c2), lambda q: (0, 0)),
            pl.BlockSpec((qb, s, 64), lambda q: (q, 0, 0)),
        ],
        out_specs=[
            pl.BlockSpec((qb, c2), lambda q: (q, 0)),
            pl.BlockSpec((qb, 64), lambda q: (q, 0)),
        ],
        out_shape=[jax.ShapeDtypeStruct((bm, c2), _F32),
                   jax.ShapeDtypeStruct((bm, 64), _F32)],
    )(z2r, st2, pb2, tr)


def _pad_rows(vecs, c):
    out = jnp.zeros((8, c), _F32)
    for i, v in enumerate(vecs):
        out = out.at[i].set(v)
    return out


def kernel(coords, features, t_embed, params1, params2):
    B, N, _ = coords.shape
    M = _M
    zpad = jnp.zeros((8 - B, N), _F32)
    cx8 = jnp.concatenate([coords[..., 0], zpad], axis=0)
    cy8 = jnp.concatenate([coords[..., 1], zpad], axis=0)
    cz8 = jnp.concatenate([coords[..., 2], zpad], axis=0)
    sx, sy, sz = _fps(cx8, cy8, cz8)
    sampled = jnp.stack([sx[:B], sy[:B], sz[:B]], axis=-1)

    featT = features.transpose(0, 2, 1).reshape(B * N, 64)
    ttab = t_embed.transpose(0, 2, 1).reshape(B * N, 64)

    (w1a, b1a, g1a, be1a), (w2a, b2a, g2a, be2a) = params1
    (w1b, b1b, g1b, be1b), (w2b, b2b, g2b, be2b) = params2
    c1a, c1b = w1a.shape[0], w1b.shape[0]
    c2a, c2b = w2a.shape[0], w2b.shape[0]

    y1t, y2t = _ytab(featT, w1a[:, 3:].T, w1b[:, 3:].T)

    cxf = coords[..., 0].reshape(B * N)
    cyf = coords[..., 1].reshape(B * N)
    czf = coords[..., 2].reshape(B * N)
    qx = sx[:B].reshape(B * M)
    qy = sy[:B].reshape(B * M)
    qz = sz[:B].reshape(B * M)
    i1, i2, rx1, ry1, rz1, rx2, ry2, rz2 = _ball_query(
        cxf, cyf, czf, qx, qy, qz)

    g1, t1 = _gather(y1t, ttab, i1, c1a)
    g2, t2 = _gather(y2t, ttab, i2, c1b)

    outs_f = []
    outs_t = []
    for (gg, rx, ry, rz, tt, s, w1, b1, g1p, be1, w2, b2, g2p, be2, c1, c2) in (
        (g1, rx1, ry1, rz1, t1, _NSAMPLES[0],
         w1a, b1a, g1a, be1a, w2a, b2a, g2a, be2a, c1a, c2a),
        (g2, rx2, ry2, rz2, t2, _NSAMPLES[1],
         w1b, b1b, g1b, be1b, w2b, b2b, g2b, be2b, c1b, c2b),
    ):
        rows = B * M * s
        wcr = _pad_rows([w1[:, 0], w1[:, 1], w1[:, 2]], c1)
        pb1 = _pad_rows([b1, g1p, be1], c1)
        pb2 = _pad_rows([b2, g2p, be2], c2)
        z1, st1 = _mlp_a(gg, rx.reshape(rows, 1), ry.reshape(rows, 1),
                         rz.reshape(rows, 1), wcr, pb1, c1)
        z2, st2 = _mlp_b(z1, st1, pb1, w2.T, pb2, c1, c2)
        f, to = _mlp_c(z2.reshape(B * M, s, c2), st2, pb2,
                       tt.reshape(B * M, s, 64), s, c2)
        outs_f.append(f.reshape(B, M, c2).transpose(0, 2, 1))
        outs_t.append(to.reshape(B, M, 64).transpose(0, 2, 1))

    out_features = jnp.concatenate(outs_f, axis=1)
    out_t_embed = jnp.concatenate(outs_t, axis=1)
    return sampled, out_features, out_t_embed


# speedup vs baseline: 496.2830x; 1.0864x over previous
import functools

import jax
import jax.numpy as jnp
from jax import lax
from jax.experimental import pallas as pl
from jax.experimental.pallas import tpu as pltpu
from jax.experimental.pallas import tpu_sc as plsc

_B = 4
_N = 8192
_M = 1024
_RADII = (0.2, 0.4)
_NSAMPLES = (16, 32)

_NUM_CORES = 2
_NUM_SUBCORES = 16
_NW = _NUM_CORES * _NUM_SUBCORES
_QPW = (_B * _M) // _NW

_F32 = jnp.float32
_I32 = jnp.int32


def _sc_mesh():
    return plsc.VectorSubcoreMesh(
        core_axis_name="c", subcore_axis_name="s",
        num_cores=_NUM_CORES, num_subcores=_NUM_SUBCORES)


_FPS_R = 8
_FPS_L = _N // _FPS_R


def _fps_body(cx_ref, cy_ref, cz_ref, sx_ref, sy_ref, sz_ref, dist_ref):
    shp = (_B, _FPS_R, _FPS_L)
    pid = (lax.broadcasted_iota(_I32, shp, 1) * _FPS_L
           + lax.broadcasted_iota(_I32, shp, 2))
    out_lanes = lax.broadcasted_iota(_I32, (_B, _M), 1)
    cx = cx_ref[...]
    cy = cy_ref[...]
    cz = cz_ref[...]

    def extract(c, sel):
        r = jnp.sum(jnp.sum(jnp.where(sel, c, 0.0), axis=2, keepdims=True),
                    axis=1, keepdims=True)
        return r

    def step(i, nxt):
        sel = pid == nxt
        px = extract(cx, sel)
        py = extract(cy, sel)
        pz = extract(cz, sel)
        m = out_lanes == i
        sx_ref[...] = jnp.where(m, px[:, 0], sx_ref[...])
        sy_ref[...] = jnp.where(m, py[:, 0], sy_ref[...])
        sz_ref[...] = jnp.where(m, pz[:, 0], sz_ref[...])
        dx = cx - px
        dy = cy - py
        dz = cz - pz
        return dx * dx + dy * dy + dz * dz

    dist_ref[...] = step(0, jnp.zeros((_B, 1, 1), _I32))

    def body(i, carry):
        dist = dist_ref[...]
        mx = jnp.max(jnp.max(dist, axis=2, keepdims=True), axis=1,
                     keepdims=True)
        nxt = jnp.min(jnp.min(jnp.where(dist == mx, pid, _N), axis=2,
                              keepdims=True), axis=1, keepdims=True)
        d2 = step(i, nxt)
        dist_ref[...] = jnp.minimum(dist, d2)
        return carry

    lax.fori_loop(1, _M, body, 0)


def _fps(cx, cy, cz):
    return pl.pallas_call(
        _fps_body,
        out_shape=[jax.ShapeDtypeStruct((_B, _M), _F32)] * 3,
        scratch_shapes=[pltpu.VMEM((_B, _FPS_R, _FPS_L), _F32)],
    )(cx, cy, cz)


def _ytab_body(x_ref, w1_ref, w2_ref, y1_ref, y2_ref):
    x = x_ref[...]
    y1_ref[...] = jnp.dot(x, w1_ref[...], preferred_element_type=_F32,
                          precision=lax.Precision.HIGHEST)
    y2_ref[...] = jnp.dot(x, w2_ref[...], preferred_element_type=_F32,
                          precision=lax.Precision.HIGHEST)


def _ytab(featT, w1t, w2t):
    rows = featT.shape[0]
    rb = 1024
    c1 = w1t.shape[1]
    c2 = w2t.shape[1]
    return pl.pallas_call(
        _ytab_body,
        grid=(rows // rb,),
        in_specs=[
            pl.BlockSpec((rb, 64), lambda i: (i, 0)),
            pl.BlockSpec((64, c1), lambda i: (0, 0)),
            pl.BlockSpec((64, c2), lambda i: (0, 0)),
        ],
        out_specs=[
            pl.BlockSpec((rb, c1), lambda i: (i, 0)),
            pl.BlockSpec((rb, c2), lambda i: (i, 0)),
        ],
        out_shape=[jax.ShapeDtypeStruct((rows, c1), _F32),
                   jax.ShapeDtypeStruct((rows, c2), _F32)],
    )(featT, w1t, w2t)


def _ball_query(cxf, cyf, czf, qx, qy, qz):
    s1, s2 = _NSAMPLES
    r1 = _B * _M * s1
    r2 = _B * _M * s2
    wpb = _NW // _B

    @functools.partial(
        pl.kernel,
        mesh=_sc_mesh(),
        compiler_params=pltpu.CompilerParams(needs_layout_passes=False),
        out_type=[
            jax.ShapeDtypeStruct((r1,), _I32),
            jax.ShapeDtypeStruct((r2,), _I32),
            jax.ShapeDtypeStruct((r1,), _F32),
            jax.ShapeDtypeStruct((r1,), _F32),
            jax.ShapeDtypeStruct((r1,), _F32),
            jax.ShapeDtypeStruct((r2,), _F32),
            jax.ShapeDtypeStruct((r2,), _F32),
            jax.ShapeDtypeStruct((r2,), _F32),
        ],
        scratch_types=[
            pltpu.VMEM((_N,), _F32),
            pltpu.VMEM((_N,), _F32),
            pltpu.VMEM((_N,), _F32),
            pltpu.VMEM((_QPW,), _F32),
            pltpu.VMEM((_QPW,), _F32),
            pltpu.VMEM((_QPW,), _F32),
            pltpu.VMEM((_QPW * s1,), _I32),
            pltpu.VMEM((_QPW * s2,), _I32),
            pltpu.VMEM((_QPW * s1,), _F32),
            pltpu.VMEM((_QPW * s1,), _F32),
            pltpu.VMEM((_QPW * s1,), _F32),
            pltpu.VMEM((_QPW * s2,), _F32),
            pltpu.VMEM((_QPW * s2,), _F32),
            pltpu.VMEM((_QPW * s2,), _F32),
            pltpu.VMEM((s1 + 16,), _I32),
            pltpu.VMEM((s2 + 16,), _I32),
        ],
    )
    def ball(cx_h, cy_h, cz_h, qx_h, qy_h, qz_h,
             i1_h, i2_h, rx1_h, ry1_h, rz1_h, rx2_h, ry2_h, rz2_h,
             cx_v, cy_v, cz_v, qx_v, qy_v, qz_v,
             i1_v, i2_v, rx1_v, ry1_v, rz1_v, rx2_v, ry2_v, rz2_v,
             buf1_v, buf2_v):
        wid = lax.axis_index("s") * _NUM_CORES + lax.axis_index("c")
        b = wid // wpb
        qbase = b * _M + (wid % wpb) * _QPW
        pltpu.sync_copy(cx_h.at[pl.ds(b * _N, _N)], cx_v)
        pltpu.sync_copy(cy_h.at[pl.ds(b * _N, _N)], cy_v)
        pltpu.sync_copy(cz_h.at[pl.ds(b * _N, _N)], cz_v)
        pltpu.sync_copy(qx_h.at[pl.ds(qbase, _QPW)], qx_v)
        pltpu.sync_copy(qy_h.at[pl.ds(qbase, _QPW)], qy_v)
        pltpu.sync_copy(qz_h.at[pl.ds(qbase, _QPW)], qz_v)

        lane = lax.iota(_I32, 16)
        zero16 = jnp.zeros((16,), _I32)

        def one_query(i, pqx, pqy, pqz, S, rad2, buf_v, iv_v, rxv, ryv, rzv):
            def cond(st):
                c, cnt = st
                return jnp.logical_and(cnt < S, c < _N // 16)

            def bodyw(st):
                c, cnt = st
                px = cx_v[pl.ds(c * 16, 16)]
                py = cy_v[pl.ds(c * 16, 16)]
                pz = cz_v[pl.ds(c * 16, 16)]
                dx = px - pqx
                dy = py - pqy
                dz = pz - pqz
                d2 = dx * dx + dy * dy + dz * dz
                msk = d2 <= rad2
                plsc.store_compressed(buf_v.at[pl.ds(cnt, 16)],
                                      lane + c * 16, mask=msk)
                npos = jnp.max(plsc.all_reduce_population_count(msk))
                return c + 1, cnt + npos

            _, cnt = lax.while_loop(cond, bodyw,
                                    (jnp.int32(0), jnp.int32(0)))
            first = plsc.load_gather(buf_v, [zero16])
            for k in range(S // 16):
                v = buf_v[pl.ds(k * 16, 16)]
                slot = lane + k * 16
                padded = jnp.where(slot < cnt, v, first)
                iv_v[pl.ds(i * S + k * 16, 16)] = padded + b * _N
                rxv[pl.ds(i * S + k * 16, 16)] = (
                    plsc.load_gather(cx_v, [padded]) - pqx)
                ryv[pl.ds(i * S + k * 16, 16)] = (
                    plsc.load_gather(cy_v, [padded]) - pqy)
                rzv[pl.ds(i * S + k * 16, 16)] = (
                    plsc.load_gather(cz_v, [padded]) - pqz)

        def qbody(i, carry):
            iv = zero16 + i
            pqx = plsc.load_gather(qx_v, [iv])
            pqy = plsc.load_gather(qy_v, [iv])
            pqz = plsc.load_gather(qz_v, [iv])
            one_query(i, pqx, pqy, pqz, s1, _RADII[0] * _RADII[0],
                      buf1_v, i1_v, rx1_v, ry1_v, rz1_v)
            one_query(i, pqx, pqy, pqz, s2, _RADII[1] * _RADII[1],
                      buf2_v, i2_v, rx2_v, ry2_v, rz2_v)
            return carry

        lax.fori_loop(0, _QPW, qbody, 0)

        pltpu.sync_copy(i1_v, i1_h.at[pl.ds(qbase * s1, _QPW * s1)])
        pltpu.sync_copy(i2_v, i2_h.at[pl.ds(qbase * s2, _QPW * s2)])
        pltpu.sync_copy(rx1_v, rx1_h.at[pl.ds(qbase * s1, _QPW * s1)])
        pltpu.sync_copy(ry1_v, ry1_h.at[pl.ds(qbase * s1, _QPW * s1)])
        pltpu.sync_copy(rz1_v, rz1_h.at[pl.ds(qbase * s1, _QPW * s1)])
        pltpu.sync_copy(rx2_v, rx2_h.at[pl.ds(qbase * s2, _QPW * s2)])
        pltpu.sync_copy(ry2_v, ry2_h.at[pl.ds(qbase * s2, _QPW * s2)])
        pltpu.sync_copy(rz2_v, rz2_h.at[pl.ds(qbase * s2, _QPW * s2)])

    return ball(cxf, cyf, czf, qx, qy, qz)


def _gather(ytab, ttab, idx, cf):
    rows = idx.shape[0]
    rpw = rows // _NW
    ch = 128
    grp = 4
    assert rpw % (ch * grp) == 0

    @functools.partial(
        pl.kernel,
        mesh=_sc_mesh(),
        compiler_params=pltpu.CompilerParams(needs_layout_passes=False,
                                             use_tc_tiling_on_sc=False),
        out_type=[jax.ShapeDtypeStruct((rows, cf), _F32),
                  jax.ShapeDtypeStruct((rows, 64), _F32)],
        scratch_types=(
            [pltpu.VMEM((rpw,), _I32)]
            + [pltpu.VMEM((ch, cf), _F32) for _ in range(grp)]
            + [pltpu.VMEM((ch, 64), _F32) for _ in range(grp)]
            + [pltpu.SemaphoreType.DMA for _ in range(2 * grp)]
        ),
    )
    def gk(ytab_h, ttab_h, idx_h, g_h, t_h, idx_v, *bufs):
        yv = bufs[:grp]
        tv = bufs[grp:2 * grp]
        sems = bufs[2 * grp:]
        wid = lax.axis_index("s") * _NUM_CORES + lax.axis_index("c")
        base = wid * rpw
        pltpu.sync_copy(idx_h.at[pl.ds(base, rpw)], idx_v)

        def body(j, carry):
            copies = []
            for g in range(grp):
                ii = idx_v.at[pl.ds((j * grp + g) * ch, ch)]
                copies.append(
                    (pltpu.async_copy(ytab_h.at[ii], yv[g], sems[2 * g]),
                     pltpu.async_copy(ttab_h.at[ii], tv[g], sems[2 * g + 1])))
            for g in range(grp):
                cy_, ct_ = copies[g]
                cy_.wait()
                ct_.wait()
                off = base + (j * grp + g) * ch
                pltpu.sync_copy(yv[g], g_h.at[pl.ds(off, ch)])
                pltpu.sync_copy(tv[g], t_h.at[pl.ds(off, ch)])
            return carry

        lax.fori_loop(0, rpw // (ch * grp), body, 0)

    return gk(ytab, ttab, idx)


def _mlp_a(g, rx, ry, rz, wcr, pb, c1):
    rows = g.shape[0]
    rb = 1024

    def body(g_ref, rx_ref, ry_ref, rz_ref, wc_ref, pb_ref, z_ref, st_ref):
        i = pl.program_id(0)
        wc = wc_ref[...]
        z = (g_ref[...]
             + rx_ref[...] * wc[0:1, :]
             + ry_ref[...] * wc[1:2, :]
             + rz_ref[...] * wc[2:3, :]
             + pb_ref[0:1, :])
        z_ref[...] = z
        s = jnp.sum(z, axis=0, keepdims=True)
        ss = jnp.sum(z * z, axis=0, keepdims=True)
        acc = jnp.concatenate([s, ss, jnp.zeros((6, c1), _F32)], axis=0)

        @pl.when(i == 0)
        def _():
            st_ref[...] = acc

        @pl.when(i != 0)
        def _():
            st_ref[...] = st_ref[...] + acc

    return pl.pallas_call(
        body,
        grid=(rows // rb,),
        in_specs=[
            pl.BlockSpec((rb, c1), lambda i: (i, 0)),
            pl.BlockSpec((rb, 1), lambda i: (i, 0)),
            pl.BlockSpec((rb, 1), lambda i: (i, 0)),
            pl.BlockSpec((rb, 1), lambda i: (i, 0)),
            pl.BlockSpec((8, c1), lambda i: (0, 0)),
            pl.BlockSpec((8, c1), lambda i: (0, 0)),
        ],
        out_specs=[
            pl.BlockSpec((rb, c1), lambda i: (i, 0)),
            pl.BlockSpec((8, c1), lambda i: (0, 0)),
        ],
        out_shape=[jax.ShapeDtypeStruct((rows, c1), _F32),
                   jax.ShapeDtypeStruct((8, c1), _F32)],
    )(g, rx, ry, rz, wcr, pb)


def _bn_coeffs(st, pb, rows):
    rinv = 1.0 / rows
    mean = st[0:1, :] * rinv
    var = st[1:2, :] * rinv - mean * mean
    a = lax.rsqrt(var + 1e-5) * pb[1:2, :]
    c = pb[2:3, :] - mean * a
    return a, c


def _mlp_b(z1, st1, pb1, w2t, pb2, c1, c2):
    rows = z1.shape[0]
    rb = 1024

    def body(z_ref, st_ref, pb1_ref, w2_ref, pb2_ref, z2_ref, st2_ref):
        i = pl.program_id(0)
        a, c = _bn_coeffs(st_ref[...], pb1_ref[...], rows)
        x1 = jnp.maximum(z_ref[...] * a + c, 0.0)
        z2 = jnp.dot(x1, w2_ref[...], preferred_element_type=_F32,
                     precision=lax.Precision.HIGHEST) + pb2_ref[0:1, :]
        z2_ref[...] = z2
        s = jnp.sum(z2, axis=0, keepdims=True)
        ss = jnp.sum(z2 * z2, axis=0, keepdims=True)
        acc = jnp.concatenate([s, ss, jnp.zeros((6, c2), _F32)], axis=0)

        @pl.when(i == 0)
        def _():
            st2_ref[...] = acc

        @pl.when(i != 0)
        def _():
            st2_ref[...] = st2_ref[...] + acc

    return pl.pallas_call(
        body,
        grid=(rows // rb,),
        in_specs=[
            pl.BlockSpec((rb, c1), lambda i: (i, 0)),
            pl.BlockSpec((8, c1), lambda i: (0, 0)),
            pl.BlockSpec((8, c1), lambda i: (0, 0)),
            pl.BlockSpec((c1, c2), lambda i: (0, 0)),
            pl.BlockSpec((8, c2), lambda i: (0, 0)),
        ],
        out_specs=[
            pl.BlockSpec((rb, c2), lambda i: (i, 0)),
            pl.BlockSpec((8, c2), lambda i: (0, 0)),
        ],
        out_shape=[jax.ShapeDtypeStruct((rows, c2), _F32),
                   jax.ShapeDtypeStruct((8, c2), _F32)],
    )(z1, st1, pb1, w2t, pb2)


def _mlp_c(z2r, st2, pb2, tr, s, c2):
    bm = z2r.shape[0]
    rows = bm * s
    qb = 64

    def body(z_ref, st_ref, pb_ref, t_ref, f_ref, to_ref):
        a, c = _bn_coeffs(st_ref[...], pb_ref[...], rows)
        x2 = jnp.maximum(z_ref[...] * a[None] + c[None], 0.0)
        f_ref[...] = jnp.max(x2, axis=1)
        to_ref[...] = jnp.max(t_ref[...], axis=1)

    return pl.pallas_call(
        body,
        grid=(bm // qb,),
        in_specs=[
            pl.BlockSpec((qb, s, c2), lambda q: (q, 0, 0)),
            pl.BlockSpec((8, c2), lambda q: (0, 0)),
            pl.BlockSpec((8, c2), lambda q: (0, 0)),
            pl.BlockSpec((qb, s, 64), lambda q: (q, 0, 0)),
        ],
        out_specs=[
            pl.BlockSpec((qb, c2), lambda q: (q, 0)),
            pl.BlockSpec((qb, 64), lambda q: (q, 0)),
        ],
        out_shape=[jax.ShapeDtypeStruct((bm, c2), _F32),
                   jax.ShapeDtypeStruct((bm, 64), _F32)],
    )(z2r, st2, pb2, tr)


def _pad_rows(vecs, c):
    out = jnp.zeros((8, c), _F32)
    for i, v in enumerate(vecs):
        out = out.at[i].set(v)
    return out


def kernel(coords, features, t_embed, params1, params2):
    B, N, _ = coords.shape
    M = _M
    fshape = (B, _FPS_R, _FPS_L)
    sx, sy, sz = _fps(coords[..., 0].reshape(fshape),
                      coords[..., 1].reshape(fshape),
                      coords[..., 2].reshape(fshape))
    sampled = jnp.stack([sx, sy, sz], axis=-1)

    featT = features.transpose(0, 2, 1).reshape(B * N, 64)
    ttab = t_embed.transpose(0, 2, 1).reshape(B * N, 64)

    (w1a, b1a, g1a, be1a), (w2a, b2a, g2a, be2a) = params1
    (w1b, b1b, g1b, be1b), (w2b, b2b, g2b, be2b) = params2
    c1a, c1b = w1a.shape[0], w1b.shape[0]
    c2a, c2b = w2a.shape[0], w2b.shape[0]

    y1t, y2t = _ytab(featT, w1a[:, 3:].T, w1b[:, 3:].T)

    cxf = coords[..., 0].reshape(B * N)
    cyf = coords[..., 1].reshape(B * N)
    czf = coords[..., 2].reshape(B * N)
    qx = sx.reshape(B * M)
    qy = sy.reshape(B * M)
    qz = sz.reshape(B * M)
    i1, i2, rx1, ry1, rz1, rx2, ry2, rz2 = _ball_query(
        cxf, cyf, czf, qx, qy, qz)

    g1, t1 = _gather(y1t, ttab, i1, c1a)
    g2, t2 = _gather(y2t, ttab, i2, c1b)

    outs_f = []
    outs_t = []
    for (gg, rx, ry, rz, tt, s, w1, b1, g1p, be1, w2, b2, g2p, be2, c1, c2) in (
        (g1, rx1, ry1, rz1, t1, _NSAMPLES[0],
         w1a, b1a, g1a, be1a, w2a, b2a, g2a, be2a, c1a, c2a),
        (g2, rx2, ry2, rz2, t2, _NSAMPLES[1],
         w1b, b1b, g1b, be1b, w2b, b2b, g2b, be2b, c1b, c2b),
    ):
        rows = B * M * s
        wcr = _pad_rows([w1[:, 0], w1[:, 1], w1[:, 2]], c1)
        pb1 = _pad_rows([b1, g1p, be1], c1)
        pb2 = _pad_rows([b2, g2p, be2], c2)
        z1, st1 = _mlp_a(gg, rx.reshape(rows, 1), ry.reshape(rows, 1),
                         rz.reshape(rows, 1), wcr, pb1, c1)
        z2, st2 = _mlp_b(z1, st1, pb1, w2.T, pb2, c1, c2)
        f, to = _mlp_c(z2.reshape(B * M, s, c2), st2, pb2,
                       tt.reshape(B * M, s, 64), s, c2)
        outs_f.append(f.reshape(B, M, c2).transpose(0, 2, 1))
        outs_t.append(to.reshape(B, M, 64).transpose(0, 2, 1))

    out_features = jnp.concatenate(outs_f, axis=1)
    out_t_embed = jnp.concatenate(outs_t, axis=1)
    return sampled, out_features, out_t_embed


# speedup vs baseline: 496.4278x; 1.0003x over previous
import functools

import jax
import jax.numpy as jnp
from jax import lax
from jax.experimental import pallas as pl
from jax.experimental.pallas import tpu as pltpu
from jax.experimental.pallas import tpu_sc as plsc

_B = 4
_N = 8192
_M = 1024
_RADII = (0.2, 0.4)
_NSAMPLES = (16, 32)

_NUM_CORES = 2
_NUM_SUBCORES = 16
_NW = _NUM_CORES * _NUM_SUBCORES
_QPW = (_B * _M) // _NW

_F32 = jnp.float32
_I32 = jnp.int32


def _sc_mesh():
    return plsc.VectorSubcoreMesh(
        core_axis_name="c", subcore_axis_name="s",
        num_cores=_NUM_CORES, num_subcores=_NUM_SUBCORES)


_FPS_R = 8
_FPS_L = _N // _FPS_R


def _fps_body(cx_ref, cy_ref, cz_ref, sx_ref, sy_ref, sz_ref, dist_ref):
    shp = (_B, _FPS_R, _FPS_L)
    pid = (lax.broadcasted_iota(_I32, shp, 1) * _FPS_L
           + lax.broadcasted_iota(_I32, shp, 2))
    out_lanes = lax.broadcasted_iota(_I32, (_B, _M), 1)
    cx = cx_ref[...]
    cy = cy_ref[...]
    cz = cz_ref[...]

    def extract(c, sel):
        r = jnp.sum(jnp.sum(jnp.where(sel, c, 0.0), axis=2, keepdims=True),
                    axis=1, keepdims=True)
        return r

    def step(i, nxt):
        sel = pid == nxt
        px = extract(cx, sel)
        py = extract(cy, sel)
        pz = extract(cz, sel)
        m = out_lanes == i
        sx_ref[...] = jnp.where(m, px[:, 0], sx_ref[...])
        sy_ref[...] = jnp.where(m, py[:, 0], sy_ref[...])
        sz_ref[...] = jnp.where(m, pz[:, 0], sz_ref[...])
        dx = cx - px
        dy = cy - py
        dz = cz - pz
        return (dx * dx + dz * dz) + dy * dy

    dist_ref[...] = step(0, jnp.zeros((_B, 1, 1), _I32))

    def body(i, carry):
        dist = dist_ref[...]
        mx = jnp.max(jnp.max(dist, axis=2, keepdims=True), axis=1,
                     keepdims=True)
        nxt = jnp.min(jnp.min(jnp.where(dist == mx, pid, _N), axis=2,
                              keepdims=True), axis=1, keepdims=True)
        d2 = step(i, nxt)
        dist_ref[...] = jnp.minimum(dist, d2)
        return carry

    lax.fori_loop(1, _M, body, 0)


def _fps(cx, cy, cz):
    return pl.pallas_call(
        _fps_body,
        out_shape=[jax.ShapeDtypeStruct((_B, _M), _F32)] * 3,
        scratch_shapes=[pltpu.VMEM((_B, _FPS_R, _FPS_L), _F32)],
    )(cx, cy, cz)


def _ytab_body(x_ref, w1_ref, w2_ref, y1_ref, y2_ref):
    x = x_ref[...]
    y1_ref[...] = jnp.dot(x, w1_ref[...], preferred_element_type=_F32,
                          precision=lax.Precision.HIGHEST)
    y2_ref[...] = jnp.dot(x, w2_ref[...], preferred_element_type=_F32,
                          precision=lax.Precision.HIGHEST)


def _ytab(featT, w1t, w2t):
    rows = featT.shape[0]
    rb = 1024
    c1 = w1t.shape[1]
    c2 = w2t.shape[1]
    return pl.pallas_call(
        _ytab_body,
        grid=(rows // rb,),
        in_specs=[
            pl.BlockSpec((rb, 64), lambda i: (i, 0)),
            pl.BlockSpec((64, c1), lambda i: (0, 0)),
            pl.BlockSpec((64, c2), lambda i: (0, 0)),
        ],
        out_specs=[
            pl.BlockSpec((rb, c1), lambda i: (i, 0)),
            pl.BlockSpec((rb, c2), lambda i: (i, 0)),
        ],
        out_shape=[jax.ShapeDtypeStruct((rows, c1), _F32),
                   jax.ShapeDtypeStruct((rows, c2), _F32)],
    )(featT, w1t, w2t)


def _ball_query(cxf, cyf, czf, qx, qy, qz):
    s1, s2 = _NSAMPLES
    r1 = _B * _M * s1
    r2 = _B * _M * s2
    wpb = _NW // _B

    @functools.partial(
        pl.kernel,
        mesh=_sc_mesh(),
        compiler_params=pltpu.CompilerParams(needs_layout_passes=False),
        out_type=[
            jax.ShapeDtypeStruct((r1,), _I32),
            jax.ShapeDtypeStruct((r2,), _I32),
            jax.ShapeDtypeStruct((r1,), _F32),
            jax.ShapeDtypeStruct((r1,), _F32),
            jax.ShapeDtypeStruct((r1,), _F32),
            jax.ShapeDtypeStruct((r2,), _F32),
            jax.ShapeDtypeStruct((r2,), _F32),
            jax.ShapeDtypeStruct((r2,), _F32),
        ],
        scratch_types=[
            pltpu.VMEM((_N,), _F32),
            pltpu.VMEM((_N,), _F32),
            pltpu.VMEM((_N,), _F32),
            pltpu.VMEM((_QPW,), _F32),
            pltpu.VMEM((_QPW,), _F32),
            pltpu.VMEM((_QPW,), _F32),
            pltpu.VMEM((_QPW * s1,), _I32),
            pltpu.VMEM((_QPW * s2,), _I32),
            pltpu.VMEM((_QPW * s1,), _F32),
            pltpu.VMEM((_QPW * s1,), _F32),
            pltpu.VMEM((_QPW * s1,), _F32),
            pltpu.VMEM((_QPW * s2,), _F32),
            pltpu.VMEM((_QPW * s2,), _F32),
            pltpu.VMEM((_QPW * s2,), _F32),
            pltpu.VMEM((s1 + 16,), _I32),
            pltpu.VMEM((s2 + 16,), _I32),
        ],
    )
    def ball(cx_h, cy_h, cz_h, qx_h, qy_h, qz_h,
             i1_h, i2_h, rx1_h, ry1_h, rz1_h, rx2_h, ry2_h, rz2_h,
             cx_v, cy_v, cz_v, qx_v, qy_v, qz_v,
             i1_v, i2_v, rx1_v, ry1_v, rz1_v, rx2_v, ry2_v, rz2_v,
             buf1_v, buf2_v):
        wid = lax.axis_index("s") * _NUM_CORES + lax.axis_index("c")
        b = wid // wpb
        qbase = b * _M + (wid % wpb) * _QPW
        pltpu.sync_copy(cx_h.at[pl.ds(b * _N, _N)], cx_v)
        pltpu.sync_copy(cy_h.at[pl.ds(b * _N, _N)], cy_v)
        pltpu.sync_copy(cz_h.at[pl.ds(b * _N, _N)], cz_v)
        pltpu.sync_copy(qx_h.at[pl.ds(qbase, _QPW)], qx_v)
        pltpu.sync_copy(qy_h.at[pl.ds(qbase, _QPW)], qy_v)
        pltpu.sync_copy(qz_h.at[pl.ds(qbase, _QPW)], qz_v)

        lane = lax.iota(_I32, 16)
        zero16 = jnp.zeros((16,), _I32)

        def one_query(i, pqx, pqy, pqz, S, rad2, buf_v, iv_v, rxv, ryv, rzv):
            def cond(st):
                c, cnt = st
                return jnp.logical_and(cnt < S, c < _N // 16)

            def bodyw(st):
                c, cnt = st
                px = cx_v[pl.ds(c * 16, 16)]
                py = cy_v[pl.ds(c * 16, 16)]
                pz = cz_v[pl.ds(c * 16, 16)]
                dx = px - pqx
                dy = py - pqy
                dz = pz - pqz
                d2 = dx * dx + dy * dy + dz * dz
                msk = d2 <= rad2
                plsc.store_compressed(buf_v.at[pl.ds(cnt, 16)],
                                      lane + c * 16, mask=msk)
                npos = jnp.max(plsc.all_reduce_population_count(msk))
                return c + 1, cnt + npos

            _, cnt = lax.while_loop(cond, bodyw,
                                    (jnp.int32(0), jnp.int32(0)))
            first = plsc.load_gather(buf_v, [zero16])
            for k in range(S // 16):
                v = buf_v[pl.ds(k * 16, 16)]
                slot = lane + k * 16
                padded = jnp.where(slot < cnt, v, first)
                iv_v[pl.ds(i * S + k * 16, 16)] = padded + b * _N
                rxv[pl.ds(i * S + k * 16, 16)] = (
                    plsc.load_gather(cx_v, [padded]) - pqx)
                ryv[pl.ds(i * S + k * 16, 16)] = (
                    plsc.load_gather(cy_v, [padded]) - pqy)
                rzv[pl.ds(i * S + k * 16, 16)] = (
                    plsc.load_gather(cz_v, [padded]) - pqz)

        def qbody(i, carry):
            iv = zero16 + i
            pqx = plsc.load_gather(qx_v, [iv])
            pqy = plsc.load_gather(qy_v, [iv])
            pqz = plsc.load_gather(qz_v, [iv])
            one_query(i, pqx, pqy, pqz, s1, _RADII[0] * _RADII[0],
                      buf1_v, i1_v, rx1_v, ry1_v, rz1_v)
            one_query(i, pqx, pqy, pqz, s2, _RADII[1] * _RADII[1],
                      buf2_v, i2_v, rx2_v, ry2_v, rz2_v)
            return carry

        lax.fori_loop(0, _QPW, qbody, 0)

        pltpu.sync_copy(i1_v, i1_h.at[pl.ds(qbase * s1, _QPW * s1)])
        pltpu.sync_copy(i2_v, i2_h.at[pl.ds(qbase * s2, _QPW * s2)])
        pltpu.sync_copy(rx1_v, rx1_h.at[pl.ds(qbase * s1, _QPW * s1)])
        pltpu.sync_copy(ry1_v, ry1_h.at[pl.ds(qbase * s1, _QPW * s1)])
        pltpu.sync_copy(rz1_v, rz1_h.at[pl.ds(qbase * s1, _QPW * s1)])
        pltpu.sync_copy(rx2_v, rx2_h.at[pl.ds(qbase * s2, _QPW * s2)])
        pltpu.sync_copy(ry2_v, ry2_h.at[pl.ds(qbase * s2, _QPW * s2)])
        pltpu.sync_copy(rz2_v, rz2_h.at[pl.ds(qbase * s2, _QPW * s2)])

    return ball(cxf, cyf, czf, qx, qy, qz)


def _gather(ytab, ttab, idx, cf):
    rows = idx.shape[0]
    rpw = rows // _NW
    ch = 128
    grp = 4
    assert rpw % (ch * grp) == 0

    @functools.partial(
        pl.kernel,
        mesh=_sc_mesh(),
        compiler_params=pltpu.CompilerParams(needs_layout_passes=False,
                                             use_tc_tiling_on_sc=False),
        out_type=[jax.ShapeDtypeStruct((rows, cf), _F32),
                  jax.ShapeDtypeStruct((rows, 64), _F32)],
        scratch_types=(
            [pltpu.VMEM((rpw,), _I32)]
            + [pltpu.VMEM((ch, cf), _F32) for _ in range(grp)]
            + [pltpu.VMEM((ch, 64), _F32) for _ in range(grp)]
            + [pltpu.SemaphoreType.DMA for _ in range(2 * grp)]
        ),
    )
    def gk(ytab_h, ttab_h, idx_h, g_h, t_h, idx_v, *bufs):
        yv = bufs[:grp]
        tv = bufs[grp:2 * grp]
        sems = bufs[2 * grp:]
        wid = lax.axis_index("s") * _NUM_CORES + lax.axis_index("c")
        base = wid * rpw
        pltpu.sync_copy(idx_h.at[pl.ds(base, rpw)], idx_v)

        def body(j, carry):
            copies = []
            for g in range(grp):
                ii = idx_v.at[pl.ds((j * grp + g) * ch, ch)]
                copies.append(
                    (pltpu.async_copy(ytab_h.at[ii], yv[g], sems[2 * g]),
                     pltpu.async_copy(ttab_h.at[ii], tv[g], sems[2 * g + 1])))
            for g in range(grp):
                cy_, ct_ = copies[g]
                cy_.wait()
                ct_.wait()
                off = base + (j * grp + g) * ch
                pltpu.sync_copy(yv[g], g_h.at[pl.ds(off, ch)])
                pltpu.sync_copy(tv[g], t_h.at[pl.ds(off, ch)])
            return carry

        lax.fori_loop(0, rpw // (ch * grp), body, 0)

    return gk(ytab, ttab, idx)


def _mlp_a(g, rx, ry, rz, wcr, pb, c1):
    rows = g.shape[0]
    rb = 1024

    def body(g_ref, rx_ref, ry_ref, rz_ref, wc_ref, pb_ref, z_ref, st_ref):
        i = pl.program_id(0)
        wc = wc_ref[...]
        z = (g_ref[...]
             + rx_ref[...] * wc[0:1, :]
             + ry_ref[...] * wc[1:2, :]
             + rz_ref[...] * wc[2:3, :]
             + pb_ref[0:1, :])
        z_ref[...] = z
        s = jnp.sum(z, axis=0, keepdims=True)
        ss = jnp.sum(z * z, axis=0, keepdims=True)
        acc = jnp.concatenate([s, ss, jnp.zeros((6, c1), _F32)], axis=0)

        @pl.when(i == 0)
        def _():
            st_ref[...] = acc

        @pl.when(i != 0)
        def _():
            st_ref[...] = st_ref[...] + acc

    return pl.pallas_call(
        body,
        grid=(rows // rb,),
        in_specs=[
            pl.BlockSpec((rb, c1), lambda i: (i, 0)),
            pl.BlockSpec((rb, 1), lambda i: (i, 0)),
            pl.BlockSpec((rb, 1), lambda i: (i, 0)),
            pl.BlockSpec((rb, 1), lambda i: (i, 0)),
            pl.BlockSpec((8, c1), lambda i: (0, 0)),
            pl.BlockSpec((8, c1), lambda i: (0, 0)),
        ],
        out_specs=[
            pl.BlockSpec((rb, c1), lambda i: (i, 0)),
            pl.BlockSpec((8, c1), lambda i: (0, 0)),
        ],
        out_shape=[jax.ShapeDtypeStruct((rows, c1), _F32),
                   jax.ShapeDtypeStruct((8, c1), _F32)],
    )(g, rx, ry, rz, wcr, pb)


def _bn_coeffs(st, pb, rows):
    rinv = 1.0 / rows
    mean = st[0:1, :] * rinv
    var = st[1:2, :] * rinv - mean * mean
    a = lax.rsqrt(var + 1e-5) * pb[1:2, :]
    c = pb[2:3, :] - mean * a
    return a, c


def _mlp_b(z1, st1, pb1, w2t, pb2, c1, c2):
    rows = z1.shape[0]
    rb = 1024

    def body(z_ref, st_ref, pb1_ref, w2_ref, pb2_ref, z2_ref, st2_ref):
        i = pl.program_id(0)
        a, c = _bn_coeffs(st_ref[...], pb1_ref[...], rows)
        x1 = jnp.maximum(z_ref[...] * a + c, 0.0)
        z2 = jnp.dot(x1, w2_ref[...], preferred_element_type=_F32,
                     precision=lax.Precision.HIGHEST) + pb2_ref[0:1, :]
        z2_ref[...] = z2
        s = jnp.sum(z2, axis=0, keepdims=True)
        ss = jnp.sum(z2 * z2, axis=0, keepdims=True)
        acc = jnp.concatenate([s, ss, jnp.zeros((6, c2), _F32)], axis=0)

        @pl.when(i == 0)
        def _():
            st2_ref[...] = acc

        @pl.when(i != 0)
        def _():
            st2_ref[...] = st2_ref[...] + acc

    return pl.pallas_call(
        body,
        grid=(rows // rb,),
        in_specs=[
            pl.BlockSpec((rb, c1), lambda i: (i, 0)),
            pl.BlockSpec((8, c1), lambda i: (0, 0)),
            pl.BlockSpec((8, c1), lambda i: (0, 0)),
            pl.BlockSpec((c1, c2), lambda i: (0, 0)),
            pl.BlockSpec((8, c2), lambda i: (0, 0)),
        ],
        out_specs=[
            pl.BlockSpec((rb, c2), lambda i: (i, 0)),
            pl.BlockSpec((8, c2), lambda i: (0, 0)),
        ],
        out_shape=[jax.ShapeDtypeStruct((rows, c2), _F32),
                   jax.ShapeDtypeStruct((8, c2), _F32)],
    )(z1, st1, pb1, w2t, pb2)


def _mlp_c(z2r, st2, pb2, tr, s, c2):
    bm = z2r.shape[0]
    rows = bm * s
    qb = 64

    def body(z_ref, st_ref, pb_ref, t_ref, f_ref, to_ref):
        a, c = _bn_coeffs(st_ref[...], pb_ref[...], rows)
        x2 = jnp.maximum(z_ref[...] * a[None] + c[None], 0.0)
        f_ref[...] = jnp.max(x2, axis=1)
        to_ref[...] = jnp.max(t_ref[...], axis=1)

    return pl.pallas_call(
        body,
        grid=(bm // qb,),
        in_specs=[
            pl.BlockSpec((qb, s, c2), lambda q: (q, 0, 0)),
            pl.BlockSpec((8, c2), lambda q: (0, 0)),
            pl.BlockSpec((8, c2), lambda q: (0, 0)),
            pl.BlockSpec((qb, s, 64), lambda q: (q, 0, 0)),
        ],
        out_specs=[
            pl.BlockSpec((qb, c2), lambda q: (q, 0)),
            pl.BlockSpec((qb, 64), lambda q: (q, 0)),
        ],
        out_shape=[jax.ShapeDtypeStruct((bm, c2), _F32),
                   jax.ShapeDtypeStruct((bm, 64), _F32)],
    )(z2r, st2, pb2, tr)


def _pad_rows(vecs, c):
    out = jnp.zeros((8, c), _F32)
    for i, v in enumerate(vecs):
        out = out.at[i].set(v)
    return out


def kernel(coords, features, t_embed, params1, params2):
    B, N, _ = coords.shape
    M = _M
    fshape = (B, _FPS_R, _FPS_L)
    sx, sy, sz = _fps(coords[..., 0].reshape(fshape),
                      coords[..., 1].reshape(fshape),
                      coords[..., 2].reshape(fshape))
    sampled = jnp.stack([sx, sy, sz], axis=-1)

    featT = features.transpose(0, 2, 1).reshape(B * N, 64)
    ttab = t_embed.transpose(0, 2, 1).reshape(B * N, 64)

    (w1a, b1a, g1a, be1a), (w2a, b2a, g2a, be2a) = params1
    (w1b, b1b, g1b, be1b), (w2b, b2b, g2b, be2b) = params2
    c1a, c1b = w1a.shape[0], w1b.shape[0]
    c2a, c2b = w2a.shape[0], w2b.shape[0]

    y1t, y2t = _ytab(featT, w1a[:, 3:].T, w1b[:, 3:].T)

    cxf = coords[..., 0].reshape(B * N)
    cyf = coords[..., 1].reshape(B * N)
    czf = coords[..., 2].reshape(B * N)
    qx = sx.reshape(B * M)
    qy = sy.reshape(B * M)
    qz = sz.reshape(B * M)
    i1, i2, rx1, ry1, rz1, rx2, ry2, rz2 = _ball_query(
        cxf, cyf, czf, qx, qy, qz)

    g1, t1 = _gather(y1t, ttab, i1, c1a)
    g2, t2 = _gather(y2t, ttab, i2, c1b)

    outs_f = []
    outs_t = []
    for (gg, rx, ry, rz, tt, s, w1, b1, g1p, be1, w2, b2, g2p, be2, c1, c2) in (
        (g1, rx1, ry1, rz1, t1, _NSAMPLES[0],
         w1a, b1a, g1a, be1a, w2a, b2a, g2a, be2a, c1a, c2a),
        (g2, rx2, ry2, rz2, t2, _NSAMPLES[1],
         w1b, b1b, g1b, be1b, w2b, b2b, g2b, be2b, c1b, c2b),
    ):
        rows = B * M * s
        wcr = _pad_rows([w1[:, 0], w1[:, 1], w1[:, 2]], c1)
        pb1 = _pad_rows([b1, g1p, be1], c1)
        pb2 = _pad_rows([b2, g2p, be2], c2)
        z1, st1 = _mlp_a(gg, rx.reshape(rows, 1), ry.reshape(rows, 1),
                         rz.reshape(rows, 1), wcr, pb1, c1)
        z2, st2 = _mlp_b(z1, st1, pb1, w2.T, pb2, c1, c2)
        f, to = _mlp_c(z2.reshape(B * M, s, c2), st2, pb2,
                       tt.reshape(B * M, s, 64), s, c2)
        outs_f.append(f.reshape(B, M, c2).transpose(0, 2, 1))
        outs_t.append(to.reshape(B, M, 64).transpose(0, 2, 1))

    out_features = jnp.concatenate(outs_f, axis=1)
    out_t_embed = jnp.concatenate(outs_t, axis=1)
    return sampled, out_features, out_t_embed


# speedup vs baseline: 527.7023x; 1.0630x over previous
import functools

import jax
import jax.numpy as jnp
from jax import lax
from jax.experimental import pallas as pl
from jax.experimental.pallas import tpu as pltpu
from jax.experimental.pallas import tpu_sc as plsc

_B = 4
_N = 8192
_M = 1024
_RADII = (0.2, 0.4)
_NSAMPLES = (16, 32)

_NUM_CORES = 2
_NUM_SUBCORES = 16
_NW = _NUM_CORES * _NUM_SUBCORES
_QPW = (_B * _M) // _NW

_F32 = jnp.float32
_I32 = jnp.int32


def _sc_mesh():
    return plsc.VectorSubcoreMesh(
        core_axis_name="c", subcore_axis_name="s",
        num_cores=_NUM_CORES, num_subcores=_NUM_SUBCORES)


_FPS_R = 8
_FPS_L = _N // _FPS_R


def _fps_body(cx_ref, cy_ref, cz_ref, sx_ref, sy_ref, sz_ref, dist_ref):
    shp = (_B, _FPS_R, _FPS_L)
    pid = (lax.broadcasted_iota(_I32, shp, 1) * _FPS_L
           + lax.broadcasted_iota(_I32, shp, 2))
    out_lanes = lax.broadcasted_iota(_I32, (_B, _M), 1)
    cx = cx_ref[...]
    cy = cy_ref[...]
    cz = cz_ref[...]

    def extract(c, sel):
        r = jnp.sum(jnp.sum(jnp.where(sel, c, 0.0), axis=2, keepdims=True),
                    axis=1, keepdims=True)
        return r

    def step(i, nxt):
        sel = pid == nxt
        px = extract(cx, sel)
        py = extract(cy, sel)
        pz = extract(cz, sel)
        m = out_lanes == i
        sx_ref[...] = jnp.where(m, px[:, 0], sx_ref[...])
        sy_ref[...] = jnp.where(m, py[:, 0], sy_ref[...])
        sz_ref[...] = jnp.where(m, pz[:, 0], sz_ref[...])
        dx = cx - px
        dy = cy - py
        dz = cz - pz
        return (dx * dx + dz * dz) + dy * dy

    dist_ref[...] = step(0, jnp.zeros((_B, 1, 1), _I32))

    def body(i, carry):
        dist = dist_ref[...]
        mx = jnp.max(jnp.max(dist, axis=2, keepdims=True), axis=1,
                     keepdims=True)
        nxt = jnp.min(jnp.min(jnp.where(dist == mx, pid, _N), axis=2,
                              keepdims=True), axis=1, keepdims=True)
        d2 = step(i, nxt)
        dist_ref[...] = jnp.minimum(dist, d2)
        return carry

    lax.fori_loop(1, _M, body, 0)


def _fps(cx, cy, cz):
    return pl.pallas_call(
        _fps_body,
        out_shape=[jax.ShapeDtypeStruct((_B, _M), _F32)] * 3,
        scratch_shapes=[pltpu.VMEM((_B, _FPS_R, _FPS_L), _F32)],
    )(cx, cy, cz)


def _ytab_body(x_ref, w1_ref, w2_ref, y1_ref, y2_ref):
    x = x_ref[...]
    y1_ref[...] = jnp.dot(x, w1_ref[...], preferred_element_type=_F32,
                          precision=lax.Precision.HIGHEST)
    y2_ref[...] = jnp.dot(x, w2_ref[...], preferred_element_type=_F32,
                          precision=lax.Precision.HIGHEST)


def _ytab(featT, w1t, w2t):
    rows = featT.shape[0]
    rb = 1024
    c1 = w1t.shape[1]
    c2 = w2t.shape[1]
    return pl.pallas_call(
        _ytab_body,
        grid=(rows // rb,),
        in_specs=[
            pl.BlockSpec((rb, 64), lambda i: (i, 0)),
            pl.BlockSpec((64, c1), lambda i: (0, 0)),
            pl.BlockSpec((64, c2), lambda i: (0, 0)),
        ],
        out_specs=[
            pl.BlockSpec((rb, c1), lambda i: (i, 0)),
            pl.BlockSpec((rb, c2), lambda i: (i, 0)),
        ],
        out_shape=[jax.ShapeDtypeStruct((rows, c1), _F32),
                   jax.ShapeDtypeStruct((rows, c2), _F32)],
    )(featT, w1t, w2t)


def _ball_query(cxf, cyf, czf, qx, qy, qz):
    s1, s2 = _NSAMPLES
    r1 = _B * _M * s1
    r2 = _B * _M * s2
    wpb = _NW // _B

    @functools.partial(
        pl.kernel,
        mesh=_sc_mesh(),
        compiler_params=pltpu.CompilerParams(needs_layout_passes=False),
        out_type=[
            jax.ShapeDtypeStruct((r1,), _I32),
            jax.ShapeDtypeStruct((r2,), _I32),
            jax.ShapeDtypeStruct((r1,), _F32),
            jax.ShapeDtypeStruct((r1,), _F32),
            jax.ShapeDtypeStruct((r1,), _F32),
            jax.ShapeDtypeStruct((r2,), _F32),
            jax.ShapeDtypeStruct((r2,), _F32),
            jax.ShapeDtypeStruct((r2,), _F32),
        ],
        scratch_types=[
            pltpu.VMEM((_N,), _F32),
            pltpu.VMEM((_N,), _F32),
            pltpu.VMEM((_N,), _F32),
            pltpu.VMEM((_QPW,), _F32),
            pltpu.VMEM((_QPW,), _F32),
            pltpu.VMEM((_QPW,), _F32),
            pltpu.VMEM((_QPW * s1,), _I32),
            pltpu.VMEM((_QPW * s2,), _I32),
            pltpu.VMEM((_QPW * s1,), _F32),
            pltpu.VMEM((_QPW * s1,), _F32),
            pltpu.VMEM((_QPW * s1,), _F32),
            pltpu.VMEM((_QPW * s2,), _F32),
            pltpu.VMEM((_QPW * s2,), _F32),
            pltpu.VMEM((_QPW * s2,), _F32),
            pltpu.VMEM((s1 + 32,), _I32),
            pltpu.VMEM((s2 + 32,), _I32),
        ],
    )
    def ball(cx_h, cy_h, cz_h, qx_h, qy_h, qz_h,
             i1_h, i2_h, rx1_h, ry1_h, rz1_h, rx2_h, ry2_h, rz2_h,
             cx_v, cy_v, cz_v, qx_v, qy_v, qz_v,
             i1_v, i2_v, rx1_v, ry1_v, rz1_v, rx2_v, ry2_v, rz2_v,
             buf1_v, buf2_v):
        wid = lax.axis_index("s") * _NUM_CORES + lax.axis_index("c")
        b = wid // wpb
        qbase = b * _M + (wid % wpb) * _QPW
        pltpu.sync_copy(cx_h.at[pl.ds(b * _N, _N)], cx_v)
        pltpu.sync_copy(cy_h.at[pl.ds(b * _N, _N)], cy_v)
        pltpu.sync_copy(cz_h.at[pl.ds(b * _N, _N)], cz_v)
        pltpu.sync_copy(qx_h.at[pl.ds(qbase, _QPW)], qx_v)
        pltpu.sync_copy(qy_h.at[pl.ds(qbase, _QPW)], qy_v)
        pltpu.sync_copy(qz_h.at[pl.ds(qbase, _QPW)], qz_v)

        lane = lax.iota(_I32, 16)
        zero16 = jnp.zeros((16,), _I32)

        def one_query(i, pqx, pqy, pqz, S, rad2, buf_v, iv_v, rxv, ryv, rzv):
            def cond(st):
                c, cnt = st
                return jnp.logical_and(cnt < S, c < _N // 32)

            def bodyw(st):
                c, cnt = st
                for half in range(2):
                    off = c * 32 + half * 16
                    px = cx_v[pl.ds(off, 16)]
                    py = cy_v[pl.ds(off, 16)]
                    pz = cz_v[pl.ds(off, 16)]
                    dx = px - pqx
                    dy = py - pqy
                    dz = pz - pqz
                    d2 = dx * dx + dy * dy + dz * dz
                    msk = d2 <= rad2
                    plsc.store_compressed(buf_v.at[pl.ds(cnt, 16)],
                                          lane + off, mask=msk)
                    cnt = cnt + plsc.all_reduce_population_count(msk)[0]
                return c + 1, cnt

            _, cnt = lax.while_loop(cond, bodyw,
                                    (jnp.int32(0), jnp.int32(0)))
            first = plsc.load_gather(buf_v, [zero16])
            for k in range(S // 16):
                v = buf_v[pl.ds(k * 16, 16)]
                slot = lane + k * 16
                padded = jnp.where(slot < cnt, v, first)
                iv_v[pl.ds(i * S + k * 16, 16)] = padded + b * _N
                rxv[pl.ds(i * S + k * 16, 16)] = (
                    plsc.load_gather(cx_v, [padded]) - pqx)
                ryv[pl.ds(i * S + k * 16, 16)] = (
                    plsc.load_gather(cy_v, [padded]) - pqy)
                rzv[pl.ds(i * S + k * 16, 16)] = (
                    plsc.load_gather(cz_v, [padded]) - pqz)

        def qbody(i, carry):
            iv = zero16 + i
            pqx = plsc.load_gather(qx_v, [iv])
            pqy = plsc.load_gather(qy_v, [iv])
            pqz = plsc.load_gather(qz_v, [iv])
            one_query(i, pqx, pqy, pqz, s1, _RADII[0] * _RADII[0],
                      buf1_v, i1_v, rx1_v, ry1_v, rz1_v)
            one_query(i, pqx, pqy, pqz, s2, _RADII[1] * _RADII[1],
                      buf2_v, i2_v, rx2_v, ry2_v, rz2_v)
            return carry

        lax.fori_loop(0, _QPW, qbody, 0)

        pltpu.sync_copy(i1_v, i1_h.at[pl.ds(qbase * s1, _QPW * s1)])
        pltpu.sync_copy(i2_v, i2_h.at[pl.ds(qbase * s2, _QPW * s2)])
        pltpu.sync_copy(rx1_v, rx1_h.at[pl.ds(qbase * s1, _QPW * s1)])
        pltpu.sync_copy(ry1_v, ry1_h.at[pl.ds(qbase * s1, _QPW * s1)])
        pltpu.sync_copy(rz1_v, rz1_h.at[pl.ds(qbase * s1, _QPW * s1)])
        pltpu.sync_copy(rx2_v, rx2_h.at[pl.ds(qbase * s2, _QPW * s2)])
        pltpu.sync_copy(ry2_v, ry2_h.at[pl.ds(qbase * s2, _QPW * s2)])
        pltpu.sync_copy(rz2_v, rz2_h.at[pl.ds(qbase * s2, _QPW * s2)])

    return ball(cxf, cyf, czf, qx, qy, qz)


def _gather(ytab, ttab, idx, cf):
    rows = idx.shape[0]
    rpw = rows // _NW
    ch = 128
    grp = 4
    assert rpw % (ch * grp) == 0

    @functools.partial(
        pl.kernel,
        mesh=_sc_mesh(),
        compiler_params=pltpu.CompilerParams(needs_layout_passes=False,
                                             use_tc_tiling_on_sc=False),
        out_type=[jax.ShapeDtypeStruct((rows, cf), _F32),
                  jax.ShapeDtypeStruct((rows, 64), _F32)],
        scratch_types=(
            [pltpu.VMEM((rpw,), _I32)]
            + [pltpu.VMEM((ch, cf), _F32) for _ in range(grp)]
            + [pltpu.VMEM((ch, 64), _F32) for _ in range(grp)]
            + [pltpu.SemaphoreType.DMA for _ in range(2 * grp)]
        ),
    )
    def gk(ytab_h, ttab_h, idx_h, g_h, t_h, idx_v, *bufs):
        yv = bufs[:grp]
        tv = bufs[grp:2 * grp]
        sems = bufs[2 * grp:]
        wid = lax.axis_index("s") * _NUM_CORES + lax.axis_index("c")
        base = wid * rpw
        pltpu.sync_copy(idx_h.at[pl.ds(base, rpw)], idx_v)

        def body(j, carry):
            copies = []
            for g in range(grp):
                ii = idx_v.at[pl.ds((j * grp + g) * ch, ch)]
                copies.append(
                    (pltpu.async_copy(ytab_h.at[ii], yv[g], sems[2 * g]),
                     pltpu.async_copy(ttab_h.at[ii], tv[g], sems[2 * g + 1])))
            for g in range(grp):
                cy_, ct_ = copies[g]
                cy_.wait()
                ct_.wait()
                off = base + (j * grp + g) * ch
                pltpu.sync_copy(yv[g], g_h.at[pl.ds(off, ch)])
                pltpu.sync_copy(tv[g], t_h.at[pl.ds(off, ch)])
            return carry

        lax.fori_loop(0, rpw // (ch * grp), body, 0)

    return gk(ytab, ttab, idx)


def _mlp_a(g, rx, ry, rz, wcr, pb, c1):
    rows = g.shape[0]
    rb = 1024

    def body(g_ref, rx_ref, ry_ref, rz_ref, wc_ref, pb_ref, z_ref, st_ref):
        i = pl.program_id(0)
        wc = wc_ref[...]
        z = (g_ref[...]
             + rx_ref[...] * wc[0:1, :]
             + ry_ref[...] * wc[1:2, :]
             + rz_ref[...] * wc[2:3, :]
             + pb_ref[0:1, :])
        z_ref[...] = z
        s = jnp.sum(z, axis=0, keepdims=True)
        ss = jnp.sum(z * z, axis=0, keepdims=True)
        acc = jnp.concatenate([s, ss, jnp.zeros((6, c1), _F32)], axis=0)

        @pl.when(i == 0)
        def _():
            st_ref[...] = acc

        @pl.when(i != 0)
        def _():
            st_ref[...] = st_ref[...] + acc

    return pl.pallas_call(
        body,
        grid=(rows // rb,),
        in_specs=[
            pl.BlockSpec((rb, c1), lambda i: (i, 0)),
            pl.BlockSpec((rb, 1), lambda i: (i, 0)),
            pl.BlockSpec((rb, 1), lambda i: (i, 0)),
            pl.BlockSpec((rb, 1), lambda i: (i, 0)),
            pl.BlockSpec((8, c1), lambda i: (0, 0)),
            pl.BlockSpec((8, c1), lambda i: (0, 0)),
        ],
        out_specs=[
            pl.BlockSpec((rb, c1), lambda i: (i, 0)),
            pl.BlockSpec((8, c1), lambda i: (0, 0)),
        ],
        out_shape=[jax.ShapeDtypeStruct((rows, c1), _F32),
                   jax.ShapeDtypeStruct((8, c1), _F32)],
    )(g, rx, ry, rz, wcr, pb)


def _bn_coeffs(st, pb, rows):
    rinv = 1.0 / rows
    mean = st[0:1, :] * rinv
    var = st[1:2, :] * rinv - mean * mean
    a = lax.rsqrt(var + 1e-5) * pb[1:2, :]
    c = pb[2:3, :] - mean * a
    return a, c


def _mlp_b(z1, st1, pb1, w2t, pb2, c1, c2):
    rows = z1.shape[0]
    rb = 1024

    def body(z_ref, st_ref, pb1_ref, w2_ref, pb2_ref, z2_ref, st2_ref):
        i = pl.program_id(0)
        a, c = _bn_coeffs(st_ref[...], pb1_ref[...], rows)
        x1 = jnp.maximum(z_ref[...] * a + c, 0.0)
        z2 = jnp.dot(x1, w2_ref[...], preferred_element_type=_F32,
                     precision=lax.Precision.HIGHEST) + pb2_ref[0:1, :]
        z2_ref[...] = z2
        s = jnp.sum(z2, axis=0, keepdims=True)
        ss = jnp.sum(z2 * z2, axis=0, keepdims=True)
        acc = jnp.concatenate([s, ss, jnp.zeros((6, c2), _F32)], axis=0)

        @pl.when(i == 0)
        def _():
            st2_ref[...] = acc

        @pl.when(i != 0)
        def _():
            st2_ref[...] = st2_ref[...] + acc

    return pl.pallas_call(
        body,
        grid=(rows // rb,),
        in_specs=[
            pl.BlockSpec((rb, c1), lambda i: (i, 0)),
            pl.BlockSpec((8, c1), lambda i: (0, 0)),
            pl.BlockSpec((8, c1), lambda i: (0, 0)),
            pl.BlockSpec((c1, c2), lambda i: (0, 0)),
            pl.BlockSpec((8, c2), lambda i: (0, 0)),
        ],
        out_specs=[
            pl.BlockSpec((rb, c2), lambda i: (i, 0)),
            pl.BlockSpec((8, c2), lambda i: (0, 0)),
        ],
        out_shape=[jax.ShapeDtypeStruct((rows, c2), _F32),
                   jax.ShapeDtypeStruct((8, c2), _F32)],
    )(z1, st1, pb1, w2t, pb2)


def _mlp_c(z2r, st2, pb2, tr, s, c2):
    bm = z2r.shape[0]
    rows = bm * s
    qb = 64

    def body(z_ref, st_ref, pb_ref, t_ref, f_ref, to_ref):
        a, c = _bn_coeffs(st_ref[...], pb_ref[...], rows)
        x2 = jnp.maximum(z_ref[...] * a[None] + c[None], 0.0)
        f_ref[...] = jnp.max(x2, axis=1)
        to_ref[...] = jnp.max(t_ref[...], axis=1)

    return pl.pallas_call(
        body,
        grid=(bm // qb,),
        in_specs=[
            pl.BlockSpec((qb, s, c2), lambda q: (q, 0, 0)),
            pl.BlockSpec((8, c2), lambda q: (0, 0)),
            pl.BlockSpec((8, c2), lambda q: (0, 0)),
            pl.BlockSpec((qb, s, 64), lambda q: (q, 0, 0)),
        ],
        out_specs=[
            pl.BlockSpec((qb, c2), lambda q: (q, 0)),
            pl.BlockSpec((qb, 64), lambda q: (q, 0)),
        ],
        out_shape=[jax.ShapeDtypeStruct((bm, c2), _F32),
                   jax.ShapeDtypeStruct((bm, 64), _F32)],
    )(z2r, st2, pb2, tr)


def _pad_rows(vecs, c):
    out = jnp.zeros((8, c), _F32)
    for i, v in enumerate(vecs):
        out = out.at[i].set(v)
    return out


def kernel(coords, features, t_embed, params1, params2):
    B, N, _ = coords.shape
    M = _M
    fshape = (B, _FPS_R, _FPS_L)
    sx, sy, sz = _fps(coords[..., 0].reshape(fshape),
                      coords[..., 1].reshape(fshape),
                      coords[..., 2].reshape(fshape))
    sampled = jnp.stack([sx, sy, sz], axis=-1)

    featT = features.transpose(0, 2, 1).reshape(B * N, 64)
    ttab = t_embed.transpose(0, 2, 1).reshape(B * N, 64)

    (w1a, b1a, g1a, be1a), (w2a, b2a, g2a, be2a) = params1
    (w1b, b1b, g1b, be1b), (w2b, b2b, g2b, be2b) = params2
    c1a, c1b = w1a.shape[0], w1b.shape[0]
    c2a, c2b = w2a.shape[0], w2b.shape[0]

    y1t, y2t = _ytab(featT, w1a[:, 3:].T, w1b[:, 3:].T)

    cxf = coords[..., 0].reshape(B * N)
    cyf = coords[..., 1].reshape(B * N)
    czf = coords[..., 2].reshape(B * N)
    qx = sx.reshape(B * M)
    qy = sy.reshape(B * M)
    qz = sz.reshape(B * M)
    i1, i2, rx1, ry1, rz1, rx2, ry2, rz2 = _ball_query(
        cxf, cyf, czf, qx, qy, qz)

    g1, t1 = _gather(y1t, ttab, i1, c1a)
    g2, t2 = _gather(y2t, ttab, i2, c1b)

    outs_f = []
    outs_t = []
    for (gg, rx, ry, rz, tt, s, w1, b1, g1p, be1, w2, b2, g2p, be2, c1, c2) in (
        (g1, rx1, ry1, rz1, t1, _NSAMPLES[0],
         w1a, b1a, g1a, be1a, w2a, b2a, g2a, be2a, c1a, c2a),
        (g2, rx2, ry2, rz2, t2, _NSAMPLES[1],
         w1b, b1b, g1b, be1b, w2b, b2b, g2b, be2b, c1b, c2b),
    ):
        rows = B * M * s
        wcr = _pad_rows([w1[:, 0], w1[:, 1], w1[:, 2]], c1)
        pb1 = _pad_rows([b1, g1p, be1], c1)
        pb2 = _pad_rows([b2, g2p, be2], c2)
        z1, st1 = _mlp_a(gg, rx.reshape(rows, 1), ry.reshape(rows, 1),
                         rz.reshape(rows, 1), wcr, pb1, c1)
        z2, st2 = _mlp_b(z1, st1, pb1, w2.T, pb2, c1, c2)
        f, to = _mlp_c(z2.reshape(B * M, s, c2), st2, pb2,
                       tt.reshape(B * M, s, 64), s, c2)
        outs_f.append(f.reshape(B, M, c2).transpose(0, 2, 1))
        outs_t.append(to.reshape(B, M, 64).transpose(0, 2, 1))

    out_features = jnp.concatenate(outs_f, axis=1)
    out_t_embed = jnp.concatenate(outs_t, axis=1)
    return sampled, out_features, out_t_embed
